# baseline (speedup 1.0000x reference)
import functools
import math

import jax
import jax.numpy as jnp
from jax.experimental import pallas as pl
from jax.experimental.pallas import tpu as pltpu

_INV_SQRT2 = 1.0 / math.sqrt(2.0)


def _gelu_exact_f32(h):
    return 0.5 * h * (1.0 + jax.lax.erf(h * jnp.float32(_INV_SQRT2)))


def _ffn_kernel(x_ref, w1_hbm, b1_ref, w2_hbm, b2_ref, o_ref,
                w1_buf, w2_buf, sem1, sem2, *, th, subtiles):
    tm = x_ref.shape[0]
    hid = w1_hbm.shape[1]
    nc = hid // th
    sub = tm // subtiles

    def start_chunk(c):
        slot = c % 2
        pltpu.make_async_copy(w1_hbm.at[:, pl.ds(c * th, th)],
                              w1_buf.at[slot], sem1.at[slot]).start()
        pltpu.make_async_copy(w2_hbm.at[pl.ds(c * th, th), :],
                              w2_buf.at[slot], sem2.at[slot]).start()

    def wait_chunk(c):
        slot = c % 2
        pltpu.make_async_copy(w1_hbm.at[:, pl.ds(0, th)],
                              w1_buf.at[slot], sem1.at[slot]).wait()
        pltpu.make_async_copy(w2_hbm.at[pl.ds(0, th), :],
                              w2_buf.at[slot], sem2.at[slot]).wait()

    start_chunk(0)
    for c in range(nc):
        if c + 1 < nc:
            start_chunk(c + 1)
        wait_chunk(c)
        slot = c % 2
        b1c = b1_ref[:, c * th:(c + 1) * th]
        for s in range(subtiles):
            rows = pl.ds(s * sub, sub)
            h = jnp.dot(x_ref[rows, :], w1_buf[slot],
                        preferred_element_type=jnp.float32)
            g = _gelu_exact_f32(h + b1c)
            part = jnp.dot(g, w2_buf[slot],
                           preferred_element_type=jnp.float32)
            if c == 0:
                o_ref[rows, :] = part + b2_ref[...]
            else:
                o_ref[rows, :] += part


@functools.partial(jax.jit, static_argnames=("tm", "th", "subtiles"))
def _mlp_forward(x, w1, b1, w2, b2, *, tm=2048, th=512, subtiles=4):
    B, N, in_feat = x.shape
    hid = w1.shape[1]
    out_feat = w2.shape[1]
    M = B * N
    x2 = x.reshape(M, in_feat)
    b1_2d = b1.reshape(1, hid)
    b2_2d = b2.reshape(1, out_feat)
    single = pl.Buffered(1)

    cost = pl.CostEstimate(
        flops=int(2 * M * (in_feat * hid + hid * out_feat)),
        transcendentals=int(M * hid),
        bytes_accessed=int(M * in_feat * 4
                           + (in_feat * hid + hid + hid * out_feat + out_feat) * 4
                           + M * out_feat * 4),
    )

    y2 = pl.pallas_call(
        functools.partial(_ffn_kernel, th=th, subtiles=subtiles),
        out_shape=jax.ShapeDtypeStruct((M, out_feat), jnp.float32),
        grid_spec=pltpu.PrefetchScalarGridSpec(
            num_scalar_prefetch=0,
            grid=(pl.cdiv(M, tm),),
            in_specs=[
                pl.BlockSpec((tm, in_feat), lambda i: (i, 0)),
                pl.BlockSpec(memory_space=pl.ANY),
                pl.BlockSpec((1, hid), lambda i: (0, 0), pipeline_mode=single),
                pl.BlockSpec(memory_space=pl.ANY),
                pl.BlockSpec((1, out_feat), lambda i: (0, 0),
                             pipeline_mode=single),
            ],
            out_specs=pl.BlockSpec((tm, out_feat), lambda i: (i, 0)),
            scratch_shapes=[
                pltpu.VMEM((2, in_feat, th), jnp.float32),
                pltpu.VMEM((2, th, out_feat), jnp.float32),
                pltpu.SemaphoreType.DMA((2,)),
                pltpu.SemaphoreType.DMA((2,)),
            ],
        ),
        compiler_params=pltpu.CompilerParams(
            dimension_semantics=("parallel",),
            vmem_limit_bytes=52 * 1024 * 1024,
        ),
        cost_estimate=cost,
    )(x2, w1, b1_2d, w2, b2_2d)

    return y2.reshape(B, N, out_feat)


def kernel(x, w1, b1, w2, b2):
    return _mlp_forward(x, w1, b1, w2, b2)
